# Initial kernel scaffold; baseline (speedup 1.0000x reference)
import jax
import jax.numpy as jnp
from jax.experimental import pallas as pl


def kernel(pos, Z, W):
    raise NotImplementedError("write your pallas kernel here")


# speedup vs baseline: 2183.0617x; 2183.0617x over previous
import math
from itertools import combinations_with_replacement

import jax
import jax.numpy as jnp
import numpy as np
from jax.experimental import pallas as pl

_SPECIES = (1, 6, 8)
_N_MAX = 8
_L_MAX = 4
_CUTOFF = 5.0
_K4PI = 4.0 * math.pi

_IU_R, _IU_C = np.triu_indices(_N_MAX)
_PS_NORM = [math.pi * math.sqrt(8.0 / (2 * l + 1)) for l in range(_L_MAX + 1)]
_PAIRS = list(combinations_with_replacement(range(len(_SPECIES)), 2))
_SQRT2 = math.sqrt(2.0)


def _rnb(x):
    xi = jax.lax.bitcast_convert_type(x, jnp.int32)
    lsb = jnp.bitwise_and(jax.lax.shift_right_logical(xi, 16), 1)
    xi = xi + 32767 + lsb
    return jax.lax.bitcast_convert_type(
        jnp.bitwise_and(xi, jnp.int32(-65536)), jnp.float32
    )


def _sph_planes(x, y, z):
    pi = math.pi
    K = _K4PI
    planes = []
    planes.append(jnp.full_like(x, K * 0.5 * math.sqrt(1.0 / pi)))
    c1 = K * math.sqrt(3.0 / (4.0 * pi))
    planes += [c1 * y, c1 * z, c1 * x]
    a2 = K * 0.5 * math.sqrt(15.0 / pi)
    b2 = K * 0.25 * math.sqrt(5.0 / pi)
    c2 = K * 0.25 * math.sqrt(15.0 / pi)
    planes += [
        a2 * x * y,
        a2 * y * z,
        b2 * (3.0 * z * z - 1.0),
        a2 * x * z,
        c2 * (x * x - y * y),
    ]
    planes += [
        K * 0.25 * math.sqrt(35.0 / (2 * pi)) * y * (3.0 * x * x - y * y),
        K * 0.5 * math.sqrt(105.0 / pi) * x * y * z,
        K * 0.25 * math.sqrt(21.0 / (2 * pi)) * y * (5.0 * z * z - 1.0),
        K * 0.25 * math.sqrt(7.0 / pi) * (5.0 * z ** 3 - 3.0 * z),
        K * 0.25 * math.sqrt(21.0 / (2 * pi)) * x * (5.0 * z * z - 1.0),
        K * 0.25 * math.sqrt(105.0 / pi) * z * (x * x - y * y),
        K * 0.25 * math.sqrt(35.0 / (2 * pi)) * x * (x * x - 3.0 * y * y),
    ]
    planes += [
        K * 0.75 * math.sqrt(35.0 / pi) * x * y * (x * x - y * y),
        K * 0.75 * math.sqrt(35.0 / (2 * pi)) * y * z * (3.0 * x * x - y * y),
        K * 0.75 * math.sqrt(5.0 / pi) * x * y * (7.0 * z * z - 1.0),
        K * 0.75 * math.sqrt(5.0 / (2 * pi)) * y * z * (7.0 * z * z - 3.0),
        K * (3.0 / 16.0) * math.sqrt(1.0 / pi)
        * (35.0 * z ** 4 - 30.0 * z * z + 3.0),
        K * 0.75 * math.sqrt(5.0 / (2 * pi)) * x * z * (7.0 * z * z - 3.0),
        K * (3.0 / 8.0) * math.sqrt(5.0 / pi) * (x * x - y * y) * (7.0 * z * z - 1.0),
        K * 0.75 * math.sqrt(35.0 / (2 * pi)) * x * z * (x * x - 3.0 * y * y),
        K * (3.0 / 16.0) * math.sqrt(35.0 / pi)
        * (x ** 4 - 6.0 * x * x * y * y + y ** 4),
    ]
    return planes


def _soap_kernel(pr_ref, pc_ref, zr_ref, zc_ref, w_ref, bas_ref, out_ref):
    f32 = jnp.float32
    pr = pr_ref[0]
    pc = pc_ref[0]
    dx = pr[0:1, :] - pc[:, 0:1]
    dy = pr[1:2, :] - pc[:, 1:2]
    dz = pr[2:3, :] - pc[:, 2:3]
    d2 = dx * dx + dy * dy + dz * dz
    inv_nrm = 1.0 / jnp.maximum(jnp.sqrt(d2), 1e-9)
    ux = dx * inv_nrm
    uy = dy * inv_nrm
    uz = dz * inv_nrm

    zr = zr_ref[0]
    zc = zc_ref[0]
    validf = (d2 < _CUTOFF * _CUTOFF).astype(f32)
    validf = validf * (zr != 0).astype(f32)
    validf = validf * (zc != 0).astype(f32)

    basq = _rnb(bas_ref[0])
    bv = [basq[a] * validf for a in range(_N_MAX)]
    wq = _rnb(w_ref[:, :])
    g = []
    for n in range(_N_MAX):
        acc = wq[n : n + 1, 0:1] * bv[0]
        for a in range(1, _N_MAX):
            acc = acc + wq[n : n + 1, a : a + 1] * bv[a]
        g.append(acc)

    ys = _sph_planes(ux, uy, uz)
    y3 = jnp.stack(ys, axis=0)

    cs = []
    for sp in _SPECIES:
        zmask = (zc == sp).astype(f32)
        rows = []
        for n in range(_N_MAX):
            a_sn = g[n] * zmask
            rows.append(jnp.sum(a_sn[None, :, :] * y3, axis=1))
        cs.append(jnp.stack(rows, axis=0))

    base = 0
    for s1, s2 in _PAIRS:
        for l in range(_L_MAX + 1):
            o, wdt = l * l, 2 * l + 1
            scale = _PS_NORM[l] * (_SQRT2 if s1 != s2 else 1.0)
            c1 = cs[s1][:, o : o + wdt, :]
            c2 = cs[s2][:, o : o + wdt, :]
            rows = []
            for n, pp in zip(_IU_R, _IU_C):
                m = scale * (_SQRT2 if n != pp else 1.0)
                rows.append(m * jnp.sum(c1[n] * c2[pp], axis=0))
            out_ref[0, pl.ds(base, 36), :] = jnp.stack(rows, axis=0)
            base += 36


def _radial_basis(pos):
    Bq, Nq, _ = pos.shape
    E = Bq * Nq * Nq
    dvec = (pos[:, None, :, :] - pos[:, :, None, :]).reshape(E, 3)
    dist = jnp.sqrt(jnp.sum(dvec * dvec, axis=1) + 1e-24)
    alpha = jnp.arange(1, _N_MAX + 1, dtype=dist.dtype)
    powers = alpha + 2.0
    norm = jnp.sqrt(_CUTOFF ** (2 * alpha + 5) / (2 * alpha + 5))
    basis = ((_CUTOFF - dist)[:, None]) ** powers[None, :]
    basis = basis / norm[None, :]
    basis = basis * (dist < _CUTOFF).astype(dist.dtype)[:, None]
    return basis.reshape(Bq, Nq, Nq, _N_MAX).transpose(0, 3, 1, 2)


def kernel(pos, Z, W):
    Bq, Nq, _ = pos.shape
    n_out = 36 * len(_PAIRS) * (_L_MAX + 1)
    pos = pos.astype(jnp.float32)
    post = jnp.transpose(pos, (0, 2, 1))
    Z = Z.astype(jnp.int32)
    zrow = Z.reshape(Bq, 1, Nq)
    zcol = Z.reshape(Bq, Nq, 1)
    bas = _radial_basis(pos)
    out = pl.pallas_call(
        _soap_kernel,
        grid=(Bq,),
        in_specs=[
            pl.BlockSpec((1, 3, Nq), lambda b: (b, 0, 0)),
            pl.BlockSpec((1, Nq, 3), lambda b: (b, 0, 0)),
            pl.BlockSpec((1, 1, Nq), lambda b: (b, 0, 0)),
            pl.BlockSpec((1, Nq, 1), lambda b: (b, 0, 0)),
            pl.BlockSpec((_N_MAX, _N_MAX), lambda b: (0, 0)),
            pl.BlockSpec((1, _N_MAX, Nq, Nq), lambda b: (b, 0, 0, 0)),
        ],
        out_specs=pl.BlockSpec((1, n_out, Nq), lambda b: (b, 0, 0)),
        out_shape=jax.ShapeDtypeStruct((Bq, n_out, Nq), jnp.float32),
    )(post, pos, zrow, zcol, W.astype(jnp.float32), bas)
    return jnp.transpose(out, (0, 2, 1))
